# speedup vs baseline: 1.9276x; 1.9276x over previous
import functools

import jax
import jax.numpy as jnp
from jax import lax
from jax.experimental import pallas as pl
from jax.experimental.pallas import tpu as pltpu
from jax.experimental.pallas import tpu_sc as plsc

N_NODES = 10000
N_EDGES = 160000
D = 256
H = 128

NCORES = 2
NTILES = 16
CHUNK = 128
NCHUNK = 160
PER_TILE = CHUNK * NCHUNK
TOTAL = PER_TILE * NTILES
ROWS_PER_TILE = 632
ACC_ROWS = ROWS_PER_TILE * NTILES
DUMMY_ROW = N_NODES


def _sc_body(xl_hbm, xr_hbm, gidx_hbm, sidx_hbm, zeros_hbm,
             accl_hbm, accr_hbm,
             acc_sh, gchunk, schunk, rows, sem):
    c = lax.axis_index("c")
    s = lax.axis_index("s")
    r0 = s * ROWS_PER_TILE

    def run(x_hbm, out_hbm):
        pltpu.sync_copy(zeros_hbm.at[pl.ds(r0, ROWS_PER_TILE)],
                        acc_sh.at[pl.ds(r0, ROWS_PER_TILE)])
        plsc.subcore_barrier()

        def chunk_step(j, carry):
            pltpu.sync_copy(gidx_hbm.at[s, j], gchunk)
            pltpu.sync_copy(sidx_hbm.at[s, j], schunk)
            pltpu.async_copy(x_hbm.at[gchunk], rows, sem).wait()
            pltpu.sync_copy(rows, acc_sh.at[schunk], add=True)
            return carry

        lax.fori_loop(0, NCHUNK, chunk_step, 0)
        plsc.subcore_barrier()
        pltpu.sync_copy(acc_sh.at[pl.ds(r0, ROWS_PER_TILE)],
                        out_hbm.at[pl.ds(r0, ROWS_PER_TILE)])

    @pl.when(c == 0)
    def _():
        run(xl_hbm, accl_hbm)

    @pl.when(c == 1)
    def _():
        run(xr_hbm, accr_hbm)


@functools.partial(
    pl.kernel,
    out_type=(jax.ShapeDtypeStruct((ACC_ROWS, H), jnp.float32),
              jax.ShapeDtypeStruct((ACC_ROWS, H), jnp.float32)),
    mesh=plsc.VectorSubcoreMesh(core_axis_name="c", subcore_axis_name="s"),
    scratch_types=[
        pltpu.VMEM_SHARED((ACC_ROWS, H), jnp.float32),
        pltpu.VMEM((CHUNK,), jnp.int32),
        pltpu.VMEM((CHUNK,), jnp.int32),
        pltpu.VMEM((CHUNK, H), jnp.float32),
        pltpu.SemaphoreType.DMA,
    ],
)
def _sc_scatter(*args):
    _sc_body(*args)


def _tc_body(x_ref, al_ref, ar_ref, w0t_ref, w1lt_ref, w1rt_ref, b_ref, o_ref):
    acc = jnp.dot(x_ref[...], w0t_ref[...], preferred_element_type=jnp.float32)
    acc += jnp.dot(al_ref[...], w1lt_ref[...], preferred_element_type=jnp.float32)
    acc += jnp.dot(ar_ref[...], w1rt_ref[...], preferred_element_type=jnp.float32)
    o_ref[...] = jnp.maximum(acc + b_ref[...], 0.0)


_ROW_BLK = 400
_GRID = N_NODES // _ROW_BLK


def kernel(features, edges, W0, b0, W1, b1):
    x = features.astype(jnp.float32)
    src = edges[0].astype(jnp.int32)
    dst = edges[1].astype(jnp.int32)

    gidx = jnp.concatenate([src, dst])
    sidx = jnp.concatenate([dst, src])
    pad = TOTAL - 2 * N_EDGES
    gidx = jnp.concatenate([gidx, jnp.zeros((pad,), jnp.int32)])
    sidx = jnp.concatenate([sidx, jnp.full((pad,), DUMMY_ROW, jnp.int32)])
    gidx3 = gidx.reshape(NTILES, NCHUNK, CHUNK)
    sidx3 = sidx.reshape(NTILES, NCHUNK, CHUNK)

    xl = x[:, :H]
    xr = x[:, H:]
    zeros = jnp.zeros((ACC_ROWS, H), jnp.float32)

    accl, accr = _sc_scatter(xl, xr, gidx3, sidx3, zeros)

    w0t = W0.T
    w1lt = W1[:, :H].T
    w1rt = W1[:, H:].T
    bsum = (b0 + b1).reshape(1, D)

    out = pl.pallas_call(
        _tc_body,
        grid=(_GRID,),
        in_specs=[
            pl.BlockSpec((_ROW_BLK, D), lambda i: (i, 0)),
            pl.BlockSpec((_ROW_BLK, H), lambda i: (i, 0)),
            pl.BlockSpec((_ROW_BLK, H), lambda i: (i, 0)),
            pl.BlockSpec((D, D), lambda i: (0, 0)),
            pl.BlockSpec((H, D), lambda i: (0, 0)),
            pl.BlockSpec((H, D), lambda i: (0, 0)),
            pl.BlockSpec((1, D), lambda i: (0, 0)),
        ],
        out_specs=pl.BlockSpec((_ROW_BLK, D), lambda i: (i, 0)),
        out_shape=jax.ShapeDtypeStruct((N_NODES, D), jnp.float32),
    )(x, accl, accr, w0t, w1lt, w1rt, bsum)
    return out


# speedup vs baseline: 2.5993x; 1.3485x over previous
import functools

import jax
import jax.numpy as jnp
from jax import lax
from jax.experimental import pallas as pl
from jax.experimental.pallas import tpu as pltpu
from jax.experimental.pallas import tpu_sc as plsc

N_NODES = 10000
N_EDGES = 160000
D = 256
H = 128

NCORES = 2
NTILES = 16
CHUNK = 128
NCHUNK = 160
PER_TILE = CHUNK * NCHUNK
TOTAL = PER_TILE * NTILES
ROWS_PER_TILE = 632
ACC_ROWS = ROWS_PER_TILE * NTILES
DUMMY_ROW = N_NODES


def _sc_body(xl_hbm, xr_hbm, gidx_hbm, sidx_hbm, zeros_hbm,
             accl_hbm, accr_hbm,
             acc_sh, gbuf0, gbuf1, sbuf0, sbuf1, rows0, rows1,
             sem0, sem1, isem0, isem1):
    c = lax.axis_index("c")
    s = lax.axis_index("s")
    r0 = s * ROWS_PER_TILE
    last = NCHUNK - 1

    def run(x_hbm, out_hbm):
        def issue_idx(j, gbuf, sbuf, isem):
            pltpu.async_copy(gidx_hbm.at[s, j], gbuf, isem)
            pltpu.async_copy(sidx_hbm.at[s, j], sbuf, isem)

        def wait_idx(gbuf, sbuf, isem):
            pltpu.make_async_copy(gidx_hbm.at[s, 0], gbuf, isem).wait()
            pltpu.make_async_copy(sidx_hbm.at[s, 0], sbuf, isem).wait()

        pltpu.sync_copy(zeros_hbm.at[pl.ds(r0, ROWS_PER_TILE)],
                        acc_sh.at[pl.ds(r0, ROWS_PER_TILE)])
        plsc.subcore_barrier()

        issue_idx(0, gbuf0, sbuf0, isem0)
        wait_idx(gbuf0, sbuf0, isem0)
        pltpu.async_copy(x_hbm.at[gbuf0], rows0, sem0)
        issue_idx(1, gbuf1, sbuf1, isem1)

        def chunk_pair(k, carry):
            a = 2 * k
            wait_idx(gbuf1, sbuf1, isem1)
            pltpu.async_copy(x_hbm.at[gbuf1], rows1, sem1)
            pltpu.make_async_copy(x_hbm.at[gbuf0], rows0, sem0).wait()
            pltpu.sync_copy(rows0, acc_sh.at[sbuf0], add=True)
            issue_idx(jnp.minimum(a + 2, last), gbuf0, sbuf0, isem0)
            wait_idx(gbuf0, sbuf0, isem0)
            pltpu.async_copy(x_hbm.at[gbuf0], rows0, sem0)
            pltpu.make_async_copy(x_hbm.at[gbuf1], rows1, sem1).wait()
            pltpu.sync_copy(rows1, acc_sh.at[sbuf1], add=True)
            issue_idx(jnp.minimum(a + 3, last), gbuf1, sbuf1, isem1)
            return carry

        lax.fori_loop(0, NCHUNK // 2, chunk_pair, 0)
        pltpu.make_async_copy(x_hbm.at[gbuf0], rows0, sem0).wait()
        wait_idx(gbuf1, sbuf1, isem1)
        plsc.subcore_barrier()
        pltpu.sync_copy(acc_sh.at[pl.ds(r0, ROWS_PER_TILE)],
                        out_hbm.at[pl.ds(r0, ROWS_PER_TILE)])

    @pl.when(c == 0)
    def _():
        run(xl_hbm, accl_hbm)

    @pl.when(c == 1)
    def _():
        run(xr_hbm, accr_hbm)


@functools.partial(
    pl.kernel,
    out_type=(jax.ShapeDtypeStruct((ACC_ROWS, H), jnp.float32),
              jax.ShapeDtypeStruct((ACC_ROWS, H), jnp.float32)),
    mesh=plsc.VectorSubcoreMesh(core_axis_name="c", subcore_axis_name="s"),
    scratch_types=[
        pltpu.VMEM_SHARED((ACC_ROWS, H), jnp.float32),
        pltpu.VMEM((CHUNK,), jnp.int32),
        pltpu.VMEM((CHUNK,), jnp.int32),
        pltpu.VMEM((CHUNK,), jnp.int32),
        pltpu.VMEM((CHUNK,), jnp.int32),
        pltpu.VMEM((CHUNK, H), jnp.float32),
        pltpu.VMEM((CHUNK, H), jnp.float32),
        pltpu.SemaphoreType.DMA,
        pltpu.SemaphoreType.DMA,
        pltpu.SemaphoreType.DMA,
        pltpu.SemaphoreType.DMA,
    ],
)
def _sc_scatter(*args):
    _sc_body(*args)


def _tc_body(x_ref, al_ref, ar_ref, w0t_ref, w1lt_ref, w1rt_ref, b_ref, o_ref):
    acc = jnp.dot(x_ref[...], w0t_ref[...], preferred_element_type=jnp.float32)
    acc += jnp.dot(al_ref[...], w1lt_ref[...], preferred_element_type=jnp.float32)
    acc += jnp.dot(ar_ref[...], w1rt_ref[...], preferred_element_type=jnp.float32)
    o_ref[...] = jnp.maximum(acc + b_ref[...], 0.0)


_ROW_BLK = 400
_GRID = N_NODES // _ROW_BLK


def kernel(features, edges, W0, b0, W1, b1):
    x = features.astype(jnp.float32)
    src = edges[0].astype(jnp.int32)
    dst = edges[1].astype(jnp.int32)

    gidx = jnp.concatenate([src, dst])
    sidx = jnp.concatenate([dst, src])
    pad = TOTAL - 2 * N_EDGES
    gidx = jnp.concatenate([gidx, jnp.zeros((pad,), jnp.int32)])
    sidx = jnp.concatenate([sidx, jnp.full((pad,), DUMMY_ROW, jnp.int32)])
    gidx3 = gidx.reshape(NTILES, NCHUNK, CHUNK)
    sidx3 = sidx.reshape(NTILES, NCHUNK, CHUNK)

    xl = x[:, :H]
    xr = x[:, H:]
    zeros = jnp.zeros((ACC_ROWS, H), jnp.float32)

    accl, accr = _sc_scatter(xl, xr, gidx3, sidx3, zeros)

    w0t = W0.T
    w1lt = W1[:, :H].T
    w1rt = W1[:, H:].T
    bsum = (b0 + b1).reshape(1, D)

    out = pl.pallas_call(
        _tc_body,
        grid=(_GRID,),
        in_specs=[
            pl.BlockSpec((_ROW_BLK, D), lambda i: (i, 0)),
            pl.BlockSpec((_ROW_BLK, H), lambda i: (i, 0)),
            pl.BlockSpec((_ROW_BLK, H), lambda i: (i, 0)),
            pl.BlockSpec((D, D), lambda i: (0, 0)),
            pl.BlockSpec((H, D), lambda i: (0, 0)),
            pl.BlockSpec((H, D), lambda i: (0, 0)),
            pl.BlockSpec((1, D), lambda i: (0, 0)),
        ],
        out_specs=pl.BlockSpec((_ROW_BLK, D), lambda i: (i, 0)),
        out_shape=jax.ShapeDtypeStruct((N_NODES, D), jnp.float32),
    )(x, accl, accr, w0t, w1lt, w1rt, bsum)
    return out
